# baseline (speedup 1.0000x reference)
import jax
import jax.numpy as jnp
from jax import lax
from jax.experimental import pallas as pl
from jax.experimental.pallas import tpu as pltpu

ALPHA = 1.0
GAMMA = 2.0
N_ROWS = 16384
N_CLS = 1000
BLK = 1024
HALF_BLOCKS = N_ROWS // (2 * BLK)


def _partial_loss(x, t):
    m = jnp.max(x, axis=1, keepdims=True)
    s = jnp.sum(jnp.exp(x - m), axis=1, keepdims=True)
    cls = lax.broadcasted_iota(jnp.int32, x.shape, 1)
    onehot = cls == t
    xt = jnp.sum(jnp.where(onehot, x, 0.0), axis=1, keepdims=True)
    logpt = (xt - m) - jnp.log(s)
    pt = jnp.exp(logpt)
    loss = -ALPHA * (1.0 - pt) * (1.0 - pt) * logpt
    return jnp.sum(loss) * (1.0 / N_ROWS)


def _focal_body(xa_ref, xb_ref, ta_ref, tb_ref, out_ref):
    i = pl.program_id(0)
    bsum = _partial_loss(xa_ref[...], ta_ref[...]) + _partial_loss(
        xb_ref[...], tb_ref[...]
    )

    @pl.when(i == 0)
    def _():
        out_ref[0, 0] = 0.0

    out_ref[0, 0] += bsum


def kernel(inputs, targets):
    t2d = targets.astype(jnp.int32).reshape(N_ROWS, 1)
    out = pl.pallas_call(
        _focal_body,
        grid=(HALF_BLOCKS,),
        in_specs=[
            pl.BlockSpec((BLK, N_CLS), lambda i: (i, 0)),
            pl.BlockSpec((BLK, N_CLS), lambda i: (i + HALF_BLOCKS, 0)),
            pl.BlockSpec((BLK, 1), lambda i: (i, 0)),
            pl.BlockSpec((BLK, 1), lambda i: (i + HALF_BLOCKS, 0)),
        ],
        out_specs=pl.BlockSpec(
            (1, 1), lambda i: (0, 0), memory_space=pltpu.SMEM
        ),
        out_shape=jax.ShapeDtypeStruct((1, 1), jnp.float32),
        compiler_params=pltpu.CompilerParams(
            dimension_semantics=("arbitrary",),
        ),
    )(inputs, inputs, t2d, t2d)
    return out[0, 0]


# speedup vs baseline: 3.2354x; 3.2354x over previous
import jax
import jax.numpy as jnp
from jax import lax
from jax.experimental import pallas as pl
from jax.experimental.pallas import tpu as pltpu

ALPHA = 1.0
GAMMA = 2.0
N_ROWS = 16384
N_CLS = 1000
BLK = 2048


def _focal_body(x_ref, t_ref, out_ref):
    i = pl.program_id(0)
    x = x_ref[...]
    t = t_ref[...]
    m = jnp.max(x, axis=0, keepdims=True)
    s = jnp.sum(jnp.exp(x - m), axis=0, keepdims=True)
    cls = lax.broadcasted_iota(jnp.int32, (N_CLS, BLK), 0)
    onehot = cls == t
    xt = jnp.sum(jnp.where(onehot, x, 0.0), axis=0, keepdims=True)
    logpt = (xt - m) - jnp.log(s)
    pt = jnp.exp(logpt)
    loss = -ALPHA * (1.0 - pt) * (1.0 - pt) * logpt
    bsum = jnp.sum(loss) * (1.0 / N_ROWS)

    @pl.when(i == 0)
    def _():
        out_ref[0, 0] = 0.0

    out_ref[0, 0] += bsum


def kernel(inputs, targets):
    xt_view = inputs.T
    t2d = targets.astype(jnp.int32).reshape(1, N_ROWS)
    out = pl.pallas_call(
        _focal_body,
        grid=(N_ROWS // BLK,),
        in_specs=[
            pl.BlockSpec((N_CLS, BLK), lambda i: (0, i)),
            pl.BlockSpec((1, BLK), lambda i: (0, i)),
        ],
        out_specs=pl.BlockSpec(
            (1, 1), lambda i: (0, 0), memory_space=pltpu.SMEM
        ),
        out_shape=jax.ShapeDtypeStruct((1, 1), jnp.float32),
        compiler_params=pltpu.CompilerParams(
            dimension_semantics=("arbitrary",),
        ),
    )(xt_view, t2d)
    return out[0, 0]
